# baseline (speedup 1.0000x reference)
import functools

import jax
import jax.numpy as jnp
from jax import lax
from jax.experimental import pallas as pl
from jax.experimental.pallas import tpu as pltpu
from jax.experimental.pallas import tpu_sc as plsc

B = 100000
D = 128
CHUNK = 128
GC = 1
GROUP = GC * CHUNK
NBUF = 6
NC = 2
NS = 16
NW = NC * NS
N_G = (B + GROUP - 1) // GROUP
LAST_START = B - GROUP
MAX_G = (N_G + NW - 1) // NW
N_EXTRA = N_G - (MAX_G - 1) * NW
BLOCK = MAX_G * GROUP
BLOCK_LAST = B - BLOCK

_mesh = plsc.VectorSubcoreMesh(core_axis_name="c", subcore_axis_name="s")


@functools.partial(
    pl.kernel,
    mesh=_mesh,
    out_type=jax.ShapeDtypeStruct((B, D), jnp.float32),
    scratch_types=[
        pltpu.VMEM((BLOCK,), jnp.int32),
        pltpu.VMEM((NBUF, GROUP, D), jnp.float32),
        pltpu.VMEM_SHARED((1024, D), jnp.float32),
    ] + [pltpu.SemaphoreType.DMA] * (2 * NBUF),
)
def _gather_sc(table_hbm, idx_hbm, out_hbm, idx_v, rows_v, table_sh, *sems):
    sem_g = sems[:NBUF]
    sem_w = sems[NBUF:]
    sub = lax.axis_index("s")
    wid = sub * NC + lax.axis_index("c")
    n_my = jnp.where(wid < N_EXTRA, MAX_G, MAX_G - 1)
    g0 = (MAX_G - 1) * wid + lax.min(wid, N_EXTRA)
    block_start = pl.multiple_of(lax.min(g0 * GROUP, BLOCK_LAST), 8)

    rows_per_sub = 1024 // NS
    tstart = pl.multiple_of(sub * rows_per_sub, 8)
    stage_t = pltpu.make_async_copy(
        table_hbm.at[pl.ds(tstart, rows_per_sub)],
        table_sh.at[pl.ds(tstart, rows_per_sub)], sem_w[0])
    stage_i = pltpu.make_async_copy(
        idx_hbm.at[pl.ds(block_start, BLOCK)], idx_v, sem_w[1])
    stage_t.start()
    stage_i.start()

    def group_start(s):
        return pl.multiple_of(lax.min((g0 + s) * GROUP, LAST_START), 8)

    def gather_copy(s, b, k, src):
        off = pl.multiple_of(group_start(s) - block_start, 8)
        return pltpu.make_async_copy(
            src.at[idx_v.at[pl.ds(off + k * CHUNK, CHUNK)]],
            rows_v.at[b, pl.ds(k * CHUNK, CHUNK)],
            sem_g[b],
        )

    stage_i.wait()
    for k in range(GC):
        gather_copy(0, 0, k, table_hbm).start()
    stage_t.wait()
    plsc.subcore_barrier()

    def write_copy(s, b):
        return pltpu.make_async_copy(
            rows_v.at[b], out_hbm.at[pl.ds(group_start(s), GROUP)], sem_w[b])

    for s in range(MAX_G + 1):
        b = s % NBUF
        if 0 < s < MAX_G:
            @pl.when(s < n_my)
            def _(s=s, b=b):
                if s >= NBUF:
                    write_copy(s - NBUF, b).wait()
                for k in range(GC):
                    gather_copy(s, b, k, table_sh).start()
        if s >= 1:
            sp, bp = s - 1, (s - 1) % NBUF
            src = table_hbm if sp == 0 else table_sh
            @pl.when(sp < n_my)
            def _(sp=sp, bp=bp, src=src):
                for k in range(GC):
                    gather_copy(sp, bp, k, src).wait()
                write_copy(sp, bp).start()

    for s in range(max(0, MAX_G - 1 - NBUF), MAX_G):
        @pl.when(jnp.logical_and(s >= n_my - NBUF, s < n_my))
        def _(s=s):
            write_copy(s, s % NBUF).wait()


def kernel(global_matrix, node_or_bond_graph_indices):
    idx = node_or_bond_graph_indices.astype(jnp.int32)
    return _gather_sc(global_matrix, idx)


# speedup vs baseline: 1.0004x; 1.0004x over previous
import functools

import jax
import jax.numpy as jnp
from jax import lax
from jax.experimental import pallas as pl
from jax.experimental.pallas import tpu as pltpu
from jax.experimental.pallas import tpu_sc as plsc

B = 100000
D = 128
CHUNK = 128
GC = 1
GROUP = GC * CHUNK
NBUF = 7
NC = 2
NS = 16
NW = NC * NS
N_G = (B + GROUP - 1) // GROUP
LAST_START = B - GROUP
MAX_G = (N_G + NW - 1) // NW
N_EXTRA = N_G - (MAX_G - 1) * NW
BLOCK = MAX_G * GROUP
BLOCK_LAST = B - BLOCK

_mesh = plsc.VectorSubcoreMesh(core_axis_name="c", subcore_axis_name="s")


@functools.partial(
    pl.kernel,
    mesh=_mesh,
    out_type=jax.ShapeDtypeStruct((B, D), jnp.float32),
    scratch_types=[
        pltpu.VMEM((BLOCK,), jnp.int32),
        pltpu.VMEM((NBUF, GROUP, D), jnp.float32),
        pltpu.VMEM_SHARED((1024, D), jnp.float32),
    ] + [pltpu.SemaphoreType.DMA] * (2 * NBUF),
)
def _gather_sc(table_hbm, idx_hbm, out_hbm, idx_v, rows_v, table_sh, *sems):
    sem_g = sems[:NBUF]
    sem_w = sems[NBUF:]
    sub = lax.axis_index("s")
    wid = sub * NC + lax.axis_index("c")
    n_my = jnp.where(wid < N_EXTRA, MAX_G, MAX_G - 1)
    g0 = (MAX_G - 1) * wid + lax.min(wid, N_EXTRA)
    block_start = pl.multiple_of(lax.min(g0 * GROUP, BLOCK_LAST), 8)

    rows_per_sub = 1024 // NS
    tstart = pl.multiple_of(sub * rows_per_sub, 8)
    stage_t = pltpu.make_async_copy(
        table_hbm.at[pl.ds(tstart, rows_per_sub)],
        table_sh.at[pl.ds(tstart, rows_per_sub)], sem_w[0])
    stage_i = pltpu.make_async_copy(
        idx_hbm.at[pl.ds(block_start, BLOCK)], idx_v, sem_w[1])
    stage_t.start()
    stage_i.start()

    def group_start(s):
        return pl.multiple_of(lax.min((g0 + s) * GROUP, LAST_START), 8)

    def gather_copy(s, b, k, src):
        off = pl.multiple_of(group_start(s) - block_start, 8)
        return pltpu.make_async_copy(
            src.at[idx_v.at[pl.ds(off + k * CHUNK, CHUNK)]],
            rows_v.at[b, pl.ds(k * CHUNK, CHUNK)],
            sem_g[b],
        )

    stage_i.wait()
    for k in range(GC):
        gather_copy(0, 0, k, table_hbm).start()
    stage_t.wait()
    plsc.subcore_barrier()

    def write_copy(s, b):
        return pltpu.make_async_copy(
            rows_v.at[b], out_hbm.at[pl.ds(group_start(s), GROUP)], sem_w[b])

    for s in range(MAX_G + 1):
        b = s % NBUF
        if 0 < s < MAX_G:
            @pl.when(s < n_my)
            def _(s=s, b=b):
                if s >= NBUF:
                    write_copy(s - NBUF, b).wait()
                for k in range(GC):
                    gather_copy(s, b, k, table_sh).start()
        if s >= 1:
            sp, bp = s - 1, (s - 1) % NBUF
            src = table_hbm if sp == 0 else table_sh
            @pl.when(sp < n_my)
            def _(sp=sp, bp=bp, src=src):
                for k in range(GC):
                    gather_copy(sp, bp, k, src).wait()
                write_copy(sp, bp).start()

    for s in range(max(0, MAX_G - 1 - NBUF), MAX_G):
        @pl.when(jnp.logical_and(s >= n_my - NBUF, s < n_my))
        def _(s=s):
            write_copy(s, s % NBUF).wait()


def kernel(global_matrix, node_or_bond_graph_indices):
    idx = node_or_bond_graph_indices.astype(jnp.int32)
    return _gather_sc(global_matrix, idx)


# speedup vs baseline: 1.0031x; 1.0027x over previous
import functools

import jax
import jax.numpy as jnp
from jax import lax
from jax.experimental import pallas as pl
from jax.experimental.pallas import tpu as pltpu
from jax.experimental.pallas import tpu_sc as plsc

B = 100000
D = 128
CHUNK = 128
GC = 1
GROUP = GC * CHUNK
NBUF = 6
NC = 2
NS = 16
NW = NC * NS
N_G = (B + GROUP - 1) // GROUP
LAST_START = B - GROUP
MAX_G = (N_G + NW - 1) // NW
N_EXTRA = N_G - (MAX_G - 1) * NW
BLOCK = MAX_G * GROUP
BLOCK_LAST = B - BLOCK

_mesh = plsc.VectorSubcoreMesh(core_axis_name="c", subcore_axis_name="s")


@functools.partial(
    pl.kernel,
    mesh=_mesh,
    out_type=jax.ShapeDtypeStruct((B, D), jnp.float32),
    scratch_types=[
        pltpu.VMEM((BLOCK,), jnp.int32),
        pltpu.VMEM((NBUF, GROUP, D), jnp.float32),
        pltpu.VMEM_SHARED((1024, D), jnp.float32),
    ] + [pltpu.SemaphoreType.DMA] * (2 * NBUF),
)
def _gather_sc(table_hbm, idx_hbm, out_hbm, idx_v, rows_v, table_sh, *sems):
    sem_g = sems[:NBUF]
    sem_w = sems[NBUF:]
    sub = lax.axis_index("s")
    wid = sub * NC + lax.axis_index("c")
    n_my = jnp.where(wid < N_EXTRA, MAX_G, MAX_G - 1)
    g0 = (MAX_G - 1) * wid + lax.min(wid, N_EXTRA)
    block_start = pl.multiple_of(lax.min(g0 * GROUP, BLOCK_LAST), 8)

    rows_per_sub = 1024 // NS
    tstart = pl.multiple_of(sub * rows_per_sub, 8)
    stage_t = pltpu.make_async_copy(
        table_hbm.at[pl.ds(tstart, rows_per_sub)],
        table_sh.at[pl.ds(tstart, rows_per_sub)], sem_w[0])
    stage_i = pltpu.make_async_copy(
        idx_hbm.at[pl.ds(block_start, BLOCK)], idx_v, sem_w[1])
    stage_t.start()
    stage_i.start()

    def group_start(s):
        return pl.multiple_of(lax.min((g0 + s) * GROUP, LAST_START), 8)

    def gather_copy(s, b, k, src):
        off = pl.multiple_of(group_start(s) - block_start, 8)
        return pltpu.make_async_copy(
            src.at[idx_v.at[pl.ds(off + k * CHUNK, CHUNK)]],
            rows_v.at[b, pl.ds(k * CHUNK, CHUNK)],
            sem_g[b],
        )

    stage_i.wait()
    for k in range(GC):
        gather_copy(0, 0, k, table_hbm).start()
    stage_t.wait()
    plsc.subcore_barrier()

    def write_copy(s, b):
        return pltpu.make_async_copy(
            rows_v.at[b], out_hbm.at[pl.ds(group_start(s), GROUP)], sem_w[b])

    for s in range(MAX_G + 1):
        b = s % NBUF
        if 0 < s < MAX_G:
            @pl.when(s < n_my)
            def _(s=s, b=b):
                if s >= NBUF:
                    write_copy(s - NBUF, b).wait()
                for k in range(GC):
                    gather_copy(s, b, k, table_sh).start()
        if s >= 1:
            sp, bp = s - 1, (s - 1) % NBUF
            src = table_hbm if sp == 0 else table_sh
            @pl.when(sp < n_my)
            def _(sp=sp, bp=bp, src=src):
                for k in range(GC):
                    gather_copy(sp, bp, k, src).wait()
                write_copy(sp, bp).start()

    for s in range(max(0, MAX_G - 1 - NBUF), MAX_G):
        @pl.when(jnp.logical_and(s >= n_my - NBUF, s < n_my))
        def _(s=s):
            write_copy(s, s % NBUF).wait()


def kernel(global_matrix, node_or_bond_graph_indices):
    idx = node_or_bond_graph_indices.astype(jnp.int32)
    return _gather_sc(global_matrix, idx)
